# baseline (speedup 1.0000x reference)
import jax
import jax.numpy as jnp
from jax.experimental import pallas as pl
from jax.experimental.pallas import tpu as pltpu

_NC = 4


def _gru_fused_kernel(emb_ref, w_ih_ref, w_hh_ref, bias_ref, b_hn_ref,
                      out_ref, hid_ref, gi_ref, h_ref):
    Tc, B, H = emb_ref.shape
    H2 = 2 * H
    c = pl.program_id(0)

    gi = jax.lax.dot_general(
        emb_ref[...], w_ih_ref[...],
        dimension_numbers=(((2,), (0,)), ((), ())),
        preferred_element_type=jnp.float32)
    gi_ref[...] = gi + bias_ref[...]

    @pl.when(c == 0)
    def _init():
        h_ref[...] = jnp.zeros_like(h_ref)

    Bh = B // 2
    b_hn = jnp.broadcast_to(b_hn_ref[...], (Bh, H))
    hA = h_ref[0:Bh]
    hB = h_ref[Bh:B]

    def half_step(h, gi_t):
        gh = jnp.dot(h.astype(jnp.bfloat16), w_hh_ref[...],
                     preferred_element_type=jnp.float32)
        r = jax.nn.sigmoid(gi_t[:, 0:H] + gh[:, 0:H])
        z = jax.nn.sigmoid(gi_t[:, H:H2] + gh[:, H:H2])
        n = jnp.tanh(gi_t[:, H2:] + r * (gh[:, H2:] + b_hn))
        return n + z * (h - n)

    for t in range(Tc):
        hA = half_step(hA, gi_ref[t, 0:Bh])
        hB = half_step(hB, gi_ref[t, Bh:B])
        out_ref[0:Bh, t, :] = hA
        out_ref[Bh:B, t, :] = hB

    h_ref[0:Bh] = hA
    h_ref[Bh:B] = hB
    hid_ref[0:Bh] = hA
    hid_ref[Bh:B] = hB


def kernel(x_ids, emb_table, w_ih, w_hh, b_ih, b_hh):
    B, T = x_ids.shape
    H = emb_table.shape[1]
    nc = _NC if T % _NC == 0 else 1
    tc = T // nc

    embedded_tm = emb_table[x_ids.T].astype(jnp.bfloat16)

    w_ih_t = w_ih.T.astype(jnp.bfloat16)
    w_hh_t = w_hh.T.astype(jnp.bfloat16)
    b_rz = jnp.concatenate([b_hh[:2 * H], jnp.zeros((H,), b_hh.dtype)])
    bias = (b_ih + b_rz).reshape(1, 3 * H)
    b_hn = b_hh[2 * H:].reshape(1, H)

    output, hidden = pl.pallas_call(
        _gru_fused_kernel,
        out_shape=(
            jax.ShapeDtypeStruct((B, T, H), jnp.float32),
            jax.ShapeDtypeStruct((B, H), jnp.float32),
        ),
        grid=(nc,),
        in_specs=[
            pl.BlockSpec((tc, B, H), lambda c: (c, 0, 0)),
            pl.BlockSpec((H, 3 * H), lambda c: (0, 0)),
            pl.BlockSpec((H, 3 * H), lambda c: (0, 0)),
            pl.BlockSpec((1, 3 * H), lambda c: (0, 0)),
            pl.BlockSpec((1, H), lambda c: (0, 0)),
        ],
        out_specs=(
            pl.BlockSpec((B, tc, H), lambda c: (0, c, 0)),
            pl.BlockSpec((B, H), lambda c: (0, 0)),
        ),
        scratch_shapes=[
            pltpu.VMEM((tc, B, 3 * H), jnp.float32),
            pltpu.VMEM((B, H), jnp.float32),
        ],
        compiler_params=pltpu.CompilerParams(
            dimension_semantics=("arbitrary",)),
    )(embedded_tm, w_ih_t, w_hh_t, bias, b_hn)

    return output, hidden.reshape(1, B, H)
